# baseline (speedup 1.0000x reference)
import jax
import jax.numpy as jnp
from jax.experimental import pallas as pl

_B, _T = 16384, 100
_GRID = 64
_BB = _B // _GRID


def _zeros_mask_kernel(o_ref):
    o_ref[...] = jnp.zeros_like(o_ref)


def kernel(inputs):
    mask = pl.pallas_call(
        _zeros_mask_kernel,
        out_shape=jax.ShapeDtypeStruct((_B, _T, 1), inputs.dtype),
        grid=(_GRID,),
        out_specs=pl.BlockSpec((_BB, _T, 1), lambda i: (i, 0, 0)),
    )()
    return (inputs, mask)


# speedup vs baseline: 3.1398x; 3.1398x over previous
import jax
import jax.numpy as jnp
from jax.experimental import pallas as pl

_B, _T = 16384, 100
_LANES = 128
_ROWS = (_B * _T) // _LANES
_BLK = 1600


def _zeros_mask_kernel(o_ref):
    o_ref[...] = jnp.zeros_like(o_ref)


def kernel(inputs):
    mask2d = pl.pallas_call(
        _zeros_mask_kernel,
        out_shape=jax.ShapeDtypeStruct((_ROWS, _LANES), inputs.dtype),
        grid=(_ROWS // _BLK,),
        out_specs=pl.BlockSpec((_BLK, _LANES), lambda i: (i, 0)),
    )()
    mask = jnp.abs(mask2d.reshape(_B, _T, 1))
    return (inputs, mask)


# speedup vs baseline: 3.2036x; 1.0203x over previous
import jax
import jax.numpy as jnp
from jax.experimental import pallas as pl

_BATCH = 16384
_TOKENS = 100
_LANES = 128
_ROWS = (_BATCH * _TOKENS) // _LANES


def _zeros_mask_kernel(o_ref):
    o_ref[...] = jnp.zeros_like(o_ref)


def kernel(inputs):
    mask2d = pl.pallas_call(
        _zeros_mask_kernel,
        out_shape=jax.ShapeDtypeStruct((_ROWS, _LANES), inputs.dtype),
        out_specs=pl.BlockSpec((_ROWS, _LANES), lambda: (0, 0)),
    )()
    mask = mask2d.reshape(_BATCH, _TOKENS, 1)
    return (inputs, mask)
